# Initial kernel scaffold; baseline (speedup 1.0000x reference)
import jax
import jax.numpy as jnp
from jax.experimental import pallas as pl


def kernel(x, edge_index, W0, b0, W1, b1, convW):
    raise NotImplementedError("write your pallas kernel here")


# speedup vs baseline: 6.6245x; 6.6245x over previous
import functools

import numpy as np
import jax
import jax.numpy as jnp
from jax import lax
from jax.experimental import pallas as pl
from jax.experimental.pallas import tpu as pltpu
from jax.experimental.pallas import tpu_sc as plsc

N = 10000
E = 320000
F_IN = 128
HID = 64
C = 64
NUM_LAYERS = 4
ALPHA = 0.1
THETA = 0.5

FW = 128
DW = 16
NC = 2
NS = 16
CHUNK = 128
CHUNKS_W = 80
HCH = CHUNKS_W // 2
E_W = CHUNK * CHUNKS_W
E_PAD = NC * NS * E_W
N_PAD = 10240
ROWS_W = N_PAD // NS
ZBLKS_W = (N_PAD // CHUNK) // NS

BLK = 256

_f32 = jnp.float32


def _fill(ref, rows, value, cols=FW):
    @pl.loop(0, rows)
    def _(i):
        for j in range(cols // 16):
            ref[i, pl.ds(16 * j, 16)] = jnp.full((16,), value, _f32)


@functools.lru_cache(maxsize=None)
def _sc_kernels():
    mesh = plsc.VectorSubcoreMesh(core_axis_name="c", subcore_axis_name="s")

    @functools.partial(
        pl.kernel,
        out_type=jax.ShapeDtypeStruct((NC, N_PAD, DW), _f32),
        mesh=mesh,
        scratch_types=[
            pltpu.VMEM((CHUNKS_W, CHUNK), jnp.int32),
            pltpu.VMEM((CHUNK, DW), _f32),
            pltpu.VMEM((CHUNK, DW), _f32),
            pltpu.VMEM_SHARED((N_PAD, DW), _f32),
        ],
    )
    def sc_deg(dst_hbm, deg_hbm, dst_v, ones_v, zero_v, acc):
        c = lax.axis_index("c")
        s = lax.axis_index("s")
        _fill(ones_v, CHUNK, 1.0, DW)
        _fill(zero_v, CHUNK, 0.0, DW)
        pltpu.sync_copy(dst_hbm.at[c, s], dst_v)

        @pl.loop(0, ZBLKS_W)
        def _(k):
            off = pl.multiple_of((s * ZBLKS_W + k) * CHUNK, CHUNK)
            pltpu.sync_copy(zero_v, acc.at[pl.ds(off, CHUNK)])

        plsc.subcore_barrier()

        @pl.loop(0, CHUNKS_W)
        def _(j):
            pltpu.sync_copy(ones_v, acc.at[dst_v.at[j]], add=True)

        plsc.subcore_barrier()
        off = pl.multiple_of(s * ROWS_W, ROWS_W)
        pltpu.sync_copy(acc.at[pl.ds(off, ROWS_W)],
                        deg_hbm.at[c, pl.ds(off, ROWS_W)])

    @functools.partial(
        pl.kernel,
        out_type=jax.ShapeDtypeStruct((NC, N_PAD, FW), _f32),
        mesh=mesh,
        scratch_types=[
            pltpu.VMEM((HCH, CHUNK), jnp.int32),
            pltpu.VMEM((HCH, CHUNK), jnp.int32),
            pltpu.VMEM((CHUNK, FW), _f32),
            pltpu.VMEM((CHUNK, FW), _f32),
            pltpu.VMEM_SHARED((N_PAD, FW), _f32),
            pltpu.SemaphoreType.DMA,
            pltpu.SemaphoreType.DMA,
        ],
    )
    def sc_spmm(hs_hbm, src_hbm, dst_hbm, out_hbm,
                src_v, dst_v, rows0, rows1, acc, sem0, sem1):
        c = lax.axis_index("c")
        s = lax.axis_index("s")
        _fill(rows0, CHUNK, 0.0)

        @pl.loop(0, ZBLKS_W)
        def _(k):
            off = pl.multiple_of((s * ZBLKS_W + k) * CHUNK, CHUNK)
            pltpu.sync_copy(rows0, acc.at[pl.ds(off, CHUNK)])

        plsc.subcore_barrier()

        for half in range(2):
            pltpu.sync_copy(src_hbm.at[c, s, pl.ds(half * HCH, HCH)], src_v)
            pltpu.sync_copy(dst_hbm.at[c, s, pl.ds(half * HCH, HCH)], dst_v)
            pltpu.async_copy(hs_hbm.at[src_v.at[0]], rows0, sem0)

            @pl.loop(0, HCH // 2)
            def _(g):
                j = g * 2
                pltpu.async_copy(hs_hbm.at[src_v.at[j + 1]], rows1, sem1)
                pltpu.make_async_copy(hs_hbm.at[src_v.at[j]], rows0,
                                      sem0).wait()
                pltpu.sync_copy(rows0, acc.at[dst_v.at[j]], add=True)

                @pl.when(j + 2 < HCH)
                def _():
                    pltpu.async_copy(hs_hbm.at[src_v.at[j + 2]], rows0, sem0)

                pltpu.make_async_copy(hs_hbm.at[src_v.at[j + 1]], rows1,
                                      sem1).wait()
                pltpu.sync_copy(rows1, acc.at[dst_v.at[j + 1]], add=True)

        plsc.subcore_barrier()
        off = pl.multiple_of(s * ROWS_W, ROWS_W)
        pltpu.sync_copy(acc.at[pl.ds(off, ROWS_W)],
                        out_hbm.at[c, pl.ds(off, ROWS_W)])

    return sc_deg, sc_spmm


def _lin0_body(x_ref, w_ref, b_ref, h_ref):
    h = jnp.dot(x_ref[...], w_ref[...], preferred_element_type=_f32)
    h_ref[...] = jnp.maximum(h + b_ref[...], 0.0)


_lin0 = pl.pallas_call(
    _lin0_body,
    grid=(N_PAD // BLK,),
    in_specs=[
        pl.BlockSpec((BLK, F_IN), lambda i: (i, 0)),
        pl.BlockSpec((F_IN, HID), lambda i: (0, 0)),
        pl.BlockSpec((1, HID), lambda i: (0, 0)),
    ],
    out_specs=pl.BlockSpec((BLK, HID), lambda i: (i, 0)),
    out_shape=jax.ShapeDtypeStruct((N_PAD, HID), _f32),
)


def _pad_fw(a):
    return jnp.concatenate([a, jnp.zeros((BLK, FW - HID), _f32)], axis=1)


def _scale_body(deg_ref, h_ref, dinv_ref, hs_ref):
    i = pl.program_id(0)
    deg = deg_ref[0, :, 0:1] + deg_ref[1, :, 0:1] + 1.0
    row = i * BLK + lax.broadcasted_iota(jnp.int32, (BLK, 1), 0)
    dinv = jnp.where(row < N, lax.rsqrt(deg), 0.0)
    dinv64 = jnp.broadcast_to(dinv, (BLK, HID))
    dinv_ref[...] = dinv64
    hs_ref[...] = _pad_fw(dinv64 * h_ref[...])


_scale = pl.pallas_call(
    _scale_body,
    grid=(N_PAD // BLK,),
    in_specs=[
        pl.BlockSpec((2, BLK, DW), lambda i: (0, i, 0)),
        pl.BlockSpec((BLK, HID), lambda i: (i, 0)),
    ],
    out_specs=[
        pl.BlockSpec((BLK, HID), lambda i: (i, 0)),
        pl.BlockSpec((BLK, FW), lambda i: (i, 0)),
    ],
    out_shape=[
        jax.ShapeDtypeStruct((N_PAD, HID), _f32),
        jax.ShapeDtypeStruct((N_PAD, FW), _f32),
    ],
)


def _eye64():
    r = lax.broadcasted_iota(jnp.int32, (HID, HID), 0)
    q = lax.broadcasted_iota(jnp.int32, (HID, HID), 1)
    return (r == q).astype(_f32)


def _res_block(parts_ref, hs_ref, x0_ref, dinv_ref, w_ref, beta):
    dinv = dinv_ref[...]
    agg = parts_ref[0, :, :HID] + parts_ref[1, :, :HID] + hs_ref[:, :HID]
    t = dinv * agg
    hr = (1.0 - ALPHA) * t + ALPHA * x0_ref[...]
    wp = beta * w_ref[...] + (1.0 - beta) * _eye64()
    return jnp.maximum(jnp.dot(hr, wp, preferred_element_type=_f32), 0.0)


def _layer_body(parts_ref, hs_ref, x0_ref, dinv_ref, w_ref, o_ref, *, beta):
    h = _res_block(parts_ref, hs_ref, x0_ref, dinv_ref, w_ref, beta)
    o_ref[...] = _pad_fw(dinv_ref[...] * h)


def _last_body(parts_ref, hs_ref, x0_ref, dinv_ref, w_ref, w1_ref, b1_ref,
               o_ref, *, beta):
    h = _res_block(parts_ref, hs_ref, x0_ref, dinv_ref, w_ref, beta)
    o = jnp.dot(h, w1_ref[...], preferred_element_type=_f32) + b1_ref[...]
    m = jnp.max(o, axis=-1, keepdims=True)
    sh = o - m
    lse = jnp.log(jnp.sum(jnp.exp(sh), axis=-1, keepdims=True))
    o_ref[...] = sh - lse


def _make_layer(beta):
    return pl.pallas_call(
        functools.partial(_layer_body, beta=beta),
        grid=(N_PAD // BLK,),
        in_specs=[
            pl.BlockSpec((2, BLK, FW), lambda i: (0, i, 0)),
            pl.BlockSpec((BLK, FW), lambda i: (i, 0)),
            pl.BlockSpec((BLK, HID), lambda i: (i, 0)),
            pl.BlockSpec((BLK, HID), lambda i: (i, 0)),
            pl.BlockSpec((HID, HID), lambda i: (0, 0)),
        ],
        out_specs=pl.BlockSpec((BLK, FW), lambda i: (i, 0)),
        out_shape=jax.ShapeDtypeStruct((N_PAD, FW), _f32),
    )


def _make_last(beta):
    return pl.pallas_call(
        functools.partial(_last_body, beta=beta),
        grid=(N_PAD // BLK,),
        in_specs=[
            pl.BlockSpec((2, BLK, FW), lambda i: (0, i, 0)),
            pl.BlockSpec((BLK, FW), lambda i: (i, 0)),
            pl.BlockSpec((BLK, HID), lambda i: (i, 0)),
            pl.BlockSpec((BLK, HID), lambda i: (i, 0)),
            pl.BlockSpec((HID, HID), lambda i: (0, 0)),
            pl.BlockSpec((HID, C), lambda i: (0, 0)),
            pl.BlockSpec((1, C), lambda i: (0, 0)),
        ],
        out_specs=pl.BlockSpec((BLK, C), lambda i: (i, 0)),
        out_shape=jax.ShapeDtypeStruct((N_PAD, C), _f32),
    )


_BETAS = [float(np.log(THETA / (l + 1) + 1.0)) for l in range(NUM_LAYERS)]
_layers = [_make_layer(b) for b in _BETAS[:-1]]
_last = _make_last(_BETAS[-1])


@jax.jit
def _forward(x, edge_index, W0, b0, W1, b1, convW):
    xp = jnp.zeros((N_PAD, F_IN), _f32).at[:N].set(x)
    src = edge_index[0].astype(jnp.int32)
    dst = edge_index[1].astype(jnp.int32)
    pad = E_PAD - E
    fill = jnp.full((pad,), N, jnp.int32)
    src_p = jnp.concatenate([src, fill]).reshape(NC, NS, CHUNKS_W, CHUNK)
    dst_p = jnp.concatenate([dst, fill]).reshape(NC, NS, CHUNKS_W, CHUNK)

    sc_deg, sc_spmm = _sc_kernels()
    degp = sc_deg(dst_p)
    h0 = _lin0(xp, W0, b0.reshape(1, HID))
    dinv64, hs = _scale(degp, h0)
    for l in range(NUM_LAYERS - 1):
        parts = sc_spmm(hs, src_p, dst_p)
        hs = _layers[l](parts, hs, h0, dinv64, convW[l])
    parts = sc_spmm(hs, src_p, dst_p)
    out = _last(parts, hs, h0, dinv64, convW[NUM_LAYERS - 1], W1,
                b1.reshape(1, C))
    return out[:N]


def kernel(x, edge_index, W0, b0, W1, b1, convW):
    return _forward(x, edge_index, W0, b0, W1, b1, convW)


# speedup vs baseline: 7.9030x; 1.1930x over previous
import functools

import numpy as np
import jax
import jax.numpy as jnp
from jax import lax
from jax.experimental import pallas as pl
from jax.experimental.pallas import tpu as pltpu
from jax.experimental.pallas import tpu_sc as plsc

N = 10000
E = 320000
F_IN = 128
HID = 64
C = 64
NUM_LAYERS = 4
ALPHA = 0.1
THETA = 0.5

FW = 128
DW = 16
NC = 2
NS = 16
CHUNK = 128
CHUNKS_W = 80
HCH = CHUNKS_W // 2
E_W = CHUNK * CHUNKS_W
E_PAD = NC * NS * E_W
N_PAD = 10240
ROWS_W = N_PAD // NS
ZBLKS_W = (N_PAD // CHUNK) // NS

BLK = 256

_f32 = jnp.float32


def _fill(ref, rows, value, cols=FW):
    @pl.loop(0, rows)
    def _(i):
        for j in range(cols // 16):
            ref[i, pl.ds(16 * j, 16)] = jnp.full((16,), value, _f32)


@functools.lru_cache(maxsize=None)
def _sc_kernels():
    mesh = plsc.VectorSubcoreMesh(core_axis_name="c", subcore_axis_name="s")

    @functools.partial(
        pl.kernel,
        out_type=jax.ShapeDtypeStruct((NC, N_PAD, DW), _f32),
        mesh=mesh,
        scratch_types=[
            pltpu.VMEM((CHUNKS_W, CHUNK), jnp.int32),
            pltpu.VMEM((CHUNK, DW), _f32),
            pltpu.VMEM((CHUNK, DW), _f32),
            pltpu.VMEM_SHARED((N_PAD, DW), _f32),
        ],
    )
    def sc_deg(dst_hbm, deg_hbm, dst_v, ones_v, zero_v, acc):
        c = lax.axis_index("c")
        s = lax.axis_index("s")
        _fill(ones_v, CHUNK, 1.0, DW)
        _fill(zero_v, CHUNK, 0.0, DW)
        pltpu.sync_copy(dst_hbm.at[c, s], dst_v)

        @pl.loop(0, ZBLKS_W)
        def _(k):
            off = pl.multiple_of((s * ZBLKS_W + k) * CHUNK, CHUNK)
            pltpu.sync_copy(zero_v, acc.at[pl.ds(off, CHUNK)])

        plsc.subcore_barrier()

        @pl.loop(0, CHUNKS_W)
        def _(j):
            pltpu.sync_copy(ones_v, acc.at[dst_v.at[j]], add=True)

        plsc.subcore_barrier()
        off = pl.multiple_of(s * ROWS_W, ROWS_W)
        pltpu.sync_copy(acc.at[pl.ds(off, ROWS_W)],
                        deg_hbm.at[c, pl.ds(off, ROWS_W)])

    @functools.partial(
        pl.kernel,
        out_type=jax.ShapeDtypeStruct((NC, N_PAD, FW), _f32),
        mesh=mesh,
        scratch_types=[
            pltpu.VMEM((HCH, CHUNK), jnp.int32),
            pltpu.VMEM((HCH, CHUNK), jnp.int32),
            pltpu.VMEM((CHUNK, FW), _f32),
            pltpu.VMEM((CHUNK, FW), _f32),
            pltpu.VMEM_SHARED((N_PAD, FW), _f32),
            pltpu.SemaphoreType.DMA,
            pltpu.SemaphoreType.DMA,
        ],
    )
    def sc_spmm(hs_hbm, src_hbm, dst_hbm, out_hbm,
                src_v, dst_v, rows0, rows1, acc, sem0, sem1):
        c = lax.axis_index("c")
        s = lax.axis_index("s")
        _fill(rows0, CHUNK, 0.0)

        @pl.loop(0, ZBLKS_W)
        def _(k):
            off = pl.multiple_of((s * ZBLKS_W + k) * CHUNK, CHUNK)
            pltpu.sync_copy(rows0, acc.at[pl.ds(off, CHUNK)])

        plsc.subcore_barrier()

        for half in range(2):
            pltpu.sync_copy(src_hbm.at[c, s, pl.ds(half * HCH, HCH)], src_v)
            pltpu.sync_copy(dst_hbm.at[c, s, pl.ds(half * HCH, HCH)], dst_v)
            hs_c = hs_hbm.at[c]
            pltpu.async_copy(hs_c.at[src_v.at[0]], rows0, sem0)

            @pl.loop(0, HCH // 2)
            def _(g):
                j = g * 2
                pltpu.async_copy(hs_c.at[src_v.at[j + 1]], rows1, sem1)
                pltpu.make_async_copy(hs_c.at[src_v.at[j]], rows0,
                                      sem0).wait()
                pltpu.sync_copy(rows0, acc.at[dst_v.at[j]], add=True)

                @pl.when(j + 2 < HCH)
                def _():
                    pltpu.async_copy(hs_c.at[src_v.at[j + 2]], rows0, sem0)

                pltpu.make_async_copy(hs_c.at[src_v.at[j + 1]], rows1,
                                      sem1).wait()
                pltpu.sync_copy(rows1, acc.at[dst_v.at[j + 1]], add=True)

        plsc.subcore_barrier()
        off = pl.multiple_of(s * ROWS_W, ROWS_W)
        pltpu.sync_copy(acc.at[pl.ds(off, ROWS_W)],
                        out_hbm.at[c, pl.ds(off, ROWS_W)])

    return sc_deg, sc_spmm


def _lin0_body(x_ref, w_ref, b_ref, h_ref):
    h = jnp.dot(x_ref[...], w_ref[...], preferred_element_type=_f32)
    h_ref[...] = jnp.maximum(h + b_ref[...], 0.0)


_lin0 = pl.pallas_call(
    _lin0_body,
    grid=(N_PAD // BLK,),
    in_specs=[
        pl.BlockSpec((BLK, F_IN), lambda i: (i, 0)),
        pl.BlockSpec((F_IN, HID), lambda i: (0, 0)),
        pl.BlockSpec((1, HID), lambda i: (0, 0)),
    ],
    out_specs=pl.BlockSpec((BLK, HID), lambda i: (i, 0)),
    out_shape=jax.ShapeDtypeStruct((N_PAD, HID), _f32),
)


def _pad_fw(a):
    return jnp.concatenate([a, jnp.zeros((BLK, FW - HID), _f32)], axis=1)


def _scale_body(deg_ref, h_ref, dinv_ref, hs_ref):
    i = pl.program_id(0)
    deg = deg_ref[0, :, 0:1] + deg_ref[1, :, 0:1] + 1.0
    row = i * BLK + lax.broadcasted_iota(jnp.int32, (BLK, 1), 0)
    dinv = jnp.where(row < N, lax.rsqrt(deg), 0.0)
    dinv64 = jnp.broadcast_to(dinv, (BLK, HID))
    dinv_ref[...] = dinv64
    hs_ref[...] = jnp.broadcast_to(_pad_fw(dinv64 * h_ref[...]), (NC, BLK, FW))


_scale = pl.pallas_call(
    _scale_body,
    grid=(N_PAD // BLK,),
    in_specs=[
        pl.BlockSpec((2, BLK, DW), lambda i: (0, i, 0)),
        pl.BlockSpec((BLK, HID), lambda i: (i, 0)),
    ],
    out_specs=[
        pl.BlockSpec((BLK, HID), lambda i: (i, 0)),
        pl.BlockSpec((NC, BLK, FW), lambda i: (0, i, 0)),
    ],
    out_shape=[
        jax.ShapeDtypeStruct((N_PAD, HID), _f32),
        jax.ShapeDtypeStruct((NC, N_PAD, FW), _f32),
    ],
)


def _eye64():
    r = lax.broadcasted_iota(jnp.int32, (HID, HID), 0)
    q = lax.broadcasted_iota(jnp.int32, (HID, HID), 1)
    return (r == q).astype(_f32)


def _res_block(parts_ref, hs_ref, x0_ref, dinv_ref, w_ref, beta):
    dinv = dinv_ref[...]
    agg = parts_ref[0, :, :HID] + parts_ref[1, :, :HID] + hs_ref[0, :, :HID]
    t = dinv * agg
    hr = (1.0 - ALPHA) * t + ALPHA * x0_ref[...]
    wp = beta * w_ref[...] + (1.0 - beta) * _eye64()
    return jnp.maximum(jnp.dot(hr, wp, preferred_element_type=_f32), 0.0)


def _layer_body(parts_ref, hs_ref, x0_ref, dinv_ref, w_ref, o_ref, *, beta):
    h = _res_block(parts_ref, hs_ref, x0_ref, dinv_ref, w_ref, beta)
    o_ref[...] = jnp.broadcast_to(_pad_fw(dinv_ref[...] * h), (NC, BLK, FW))


def _last_body(parts_ref, hs_ref, x0_ref, dinv_ref, w_ref, w1_ref, b1_ref,
               o_ref, *, beta):
    h = _res_block(parts_ref, hs_ref, x0_ref, dinv_ref, w_ref, beta)
    o = jnp.dot(h, w1_ref[...], preferred_element_type=_f32) + b1_ref[...]
    m = jnp.max(o, axis=-1, keepdims=True)
    sh = o - m
    lse = jnp.log(jnp.sum(jnp.exp(sh), axis=-1, keepdims=True))
    o_ref[...] = sh - lse


def _make_layer(beta):
    return pl.pallas_call(
        functools.partial(_layer_body, beta=beta),
        grid=(N_PAD // BLK,),
        in_specs=[
            pl.BlockSpec((2, BLK, FW), lambda i: (0, i, 0)),
            pl.BlockSpec((NC, BLK, FW), lambda i: (0, i, 0)),
            pl.BlockSpec((BLK, HID), lambda i: (i, 0)),
            pl.BlockSpec((BLK, HID), lambda i: (i, 0)),
            pl.BlockSpec((HID, HID), lambda i: (0, 0)),
        ],
        out_specs=pl.BlockSpec((NC, BLK, FW), lambda i: (0, i, 0)),
        out_shape=jax.ShapeDtypeStruct((NC, N_PAD, FW), _f32),
    )


def _make_last(beta):
    return pl.pallas_call(
        functools.partial(_last_body, beta=beta),
        grid=(N_PAD // BLK,),
        in_specs=[
            pl.BlockSpec((2, BLK, FW), lambda i: (0, i, 0)),
            pl.BlockSpec((NC, BLK, FW), lambda i: (0, i, 0)),
            pl.BlockSpec((BLK, HID), lambda i: (i, 0)),
            pl.BlockSpec((BLK, HID), lambda i: (i, 0)),
            pl.BlockSpec((HID, HID), lambda i: (0, 0)),
            pl.BlockSpec((HID, C), lambda i: (0, 0)),
            pl.BlockSpec((1, C), lambda i: (0, 0)),
        ],
        out_specs=pl.BlockSpec((BLK, C), lambda i: (i, 0)),
        out_shape=jax.ShapeDtypeStruct((N_PAD, C), _f32),
    )


_BETAS = [float(np.log(THETA / (l + 1) + 1.0)) for l in range(NUM_LAYERS)]
_layers = [_make_layer(b) for b in _BETAS[:-1]]
_last = _make_last(_BETAS[-1])


@jax.jit
def _forward(x, edge_index, W0, b0, W1, b1, convW):
    xp = jnp.zeros((N_PAD, F_IN), _f32).at[:N].set(x)
    src = edge_index[0].astype(jnp.int32)
    dst = edge_index[1].astype(jnp.int32)
    pad = E_PAD - E
    fill = jnp.full((pad,), N, jnp.int32)
    src_p = jnp.concatenate([src, fill]).reshape(NC, NS, CHUNKS_W, CHUNK)
    dst_p = jnp.concatenate([dst, fill]).reshape(NC, NS, CHUNKS_W, CHUNK)

    sc_deg, sc_spmm = _sc_kernels()
    degp = sc_deg(dst_p)
    h0 = _lin0(xp, W0, b0.reshape(1, HID))
    dinv64, hs = _scale(degp, h0)
    for l in range(NUM_LAYERS - 1):
        parts = sc_spmm(hs, src_p, dst_p)
        hs = _layers[l](parts, hs, h0, dinv64, convW[l])
    parts = sc_spmm(hs, src_p, dst_p)
    out = _last(parts, hs, h0, dinv64, convW[NUM_LAYERS - 1], W1,
                b1.reshape(1, C))
    return out[:N]


def kernel(x, edge_index, W0, b0, W1, b1, convW):
    return _forward(x, edge_index, W0, b0, W1, b1, convW)


# speedup vs baseline: 7.9040x; 1.0001x over previous
import functools

import numpy as np
import jax
import jax.numpy as jnp
from jax import lax
from jax.experimental import pallas as pl
from jax.experimental.pallas import tpu as pltpu
from jax.experimental.pallas import tpu_sc as plsc

N = 10000
E = 320000
F_IN = 128
HID = 64
C = 64
NUM_LAYERS = 4
ALPHA = 0.1
THETA = 0.5

FW = 128
DW = 16
NC = 2
NS = 16
CHUNK = 128
CHUNKS_W = 80
HCH = CHUNKS_W // 2
E_W = CHUNK * CHUNKS_W
E_PAD = NC * NS * E_W
N_PAD = 10240
ROWS_W = N_PAD // NS
ZBLKS_W = (N_PAD // CHUNK) // NS

BLK = 256

_f32 = jnp.float32


def _fill(ref, rows, value, cols=FW):
    @pl.loop(0, rows)
    def _(i):
        for j in range(cols // 16):
            ref[i, pl.ds(16 * j, 16)] = jnp.full((16,), value, _f32)


@functools.lru_cache(maxsize=None)
def _sc_kernels():
    mesh = plsc.VectorSubcoreMesh(core_axis_name="c", subcore_axis_name="s")

    @functools.partial(
        pl.kernel,
        out_type=jax.ShapeDtypeStruct((NC, N_PAD, DW), _f32),
        mesh=mesh,
        scratch_types=[
            pltpu.VMEM((CHUNKS_W, CHUNK), jnp.int32),
            pltpu.VMEM((CHUNK, DW), _f32),
            pltpu.VMEM((CHUNK, DW), _f32),
            pltpu.VMEM_SHARED((N_PAD, DW), _f32),
        ],
    )
    def sc_deg(dst_hbm, deg_hbm, dst_v, ones_v, zero_v, acc):
        c = lax.axis_index("c")
        s = lax.axis_index("s")
        _fill(ones_v, CHUNK, 1.0, DW)
        _fill(zero_v, CHUNK, 0.0, DW)
        pltpu.sync_copy(dst_hbm.at[c, s], dst_v)

        @pl.loop(0, ZBLKS_W)
        def _(k):
            off = pl.multiple_of((s * ZBLKS_W + k) * CHUNK, CHUNK)
            pltpu.sync_copy(zero_v, acc.at[pl.ds(off, CHUNK)])

        plsc.subcore_barrier()

        @pl.loop(0, CHUNKS_W)
        def _(j):
            pltpu.sync_copy(ones_v, acc.at[dst_v.at[j]], add=True)

        plsc.subcore_barrier()
        off = pl.multiple_of(s * ROWS_W, ROWS_W)
        pltpu.sync_copy(acc.at[pl.ds(off, ROWS_W)],
                        deg_hbm.at[c, pl.ds(off, ROWS_W)])

    @functools.partial(
        pl.kernel,
        out_type=jax.ShapeDtypeStruct((NC, N_PAD, FW), _f32),
        mesh=mesh,
        scratch_types=[
            pltpu.VMEM((HCH, CHUNK), jnp.int32),
            pltpu.VMEM((HCH, CHUNK), jnp.int32),
            pltpu.VMEM((CHUNK, FW), _f32),
            pltpu.VMEM((CHUNK, FW), _f32),
            pltpu.VMEM_SHARED((N_PAD, FW), _f32),
            pltpu.SemaphoreType.DMA,
            pltpu.SemaphoreType.DMA,
        ],
    )
    def sc_spmm(hs_hbm, src_hbm, dst_hbm, out_hbm,
                src_v, dst_v, rows0, rows1, acc, sem0, sem1):
        c = lax.axis_index("c")
        s = lax.axis_index("s")
        _fill(rows0, CHUNK, 0.0)

        @pl.loop(0, ZBLKS_W)
        def _(k):
            off = pl.multiple_of((s * ZBLKS_W + k) * CHUNK, CHUNK)
            pltpu.sync_copy(rows0, acc.at[pl.ds(off, CHUNK)])

        plsc.subcore_barrier()

        hs_c = hs_hbm.at[c]
        for half in range(2):
            pltpu.sync_copy(src_hbm.at[c, s, pl.ds(half * HCH, HCH)], src_v)
            pltpu.sync_copy(dst_hbm.at[c, s, pl.ds(half * HCH, HCH)], dst_v)
            pltpu.async_copy(hs_c.at[src_v.at[0]], rows0, sem0)

            @pl.loop(0, HCH // 2)
            def _(g):
                j = g * 2
                pltpu.async_copy(hs_c.at[src_v.at[j + 1]], rows1, sem1)
                pltpu.make_async_copy(hs_c.at[src_v.at[j]], rows0,
                                      sem0).wait()
                pltpu.sync_copy(rows0, acc.at[dst_v.at[j]], add=True)

                @pl.when(j + 2 < HCH)
                def _():
                    pltpu.async_copy(hs_c.at[src_v.at[j + 2]], rows0, sem0)

                pltpu.make_async_copy(hs_c.at[src_v.at[j + 1]], rows1,
                                      sem1).wait()
                pltpu.sync_copy(rows1, acc.at[dst_v.at[j + 1]], add=True)

        plsc.subcore_barrier()
        off = pl.multiple_of(s * ROWS_W, ROWS_W)
        pltpu.sync_copy(acc.at[pl.ds(off, ROWS_W)],
                        out_hbm.at[c, pl.ds(off, ROWS_W)])

    return sc_deg, sc_spmm


def _lin0_body(x_ref, w_ref, b_ref, h_ref):
    h = jnp.dot(x_ref[...], w_ref[...], preferred_element_type=_f32)
    h_ref[...] = jnp.maximum(h + b_ref[...], 0.0)


_lin0 = pl.pallas_call(
    _lin0_body,
    grid=(N_PAD // BLK,),
    in_specs=[
        pl.BlockSpec((BLK, F_IN), lambda i: (i, 0)),
        pl.BlockSpec((F_IN, HID), lambda i: (0, 0)),
        pl.BlockSpec((1, HID), lambda i: (0, 0)),
    ],
    out_specs=pl.BlockSpec((BLK, HID), lambda i: (i, 0)),
    out_shape=jax.ShapeDtypeStruct((N_PAD, HID), _f32),
)


def _pad_fw(a):
    return jnp.concatenate([a, jnp.zeros((BLK, FW - HID), _f32)], axis=1)


def _scale_body(deg_ref, h_ref, dinv_ref, hs_ref):
    i = pl.program_id(0)
    deg = deg_ref[0, :, 0:1] + deg_ref[1, :, 0:1] + 1.0
    row = i * BLK + lax.broadcasted_iota(jnp.int32, (BLK, 1), 0)
    dinv = jnp.where(row < N, lax.rsqrt(deg), 0.0)
    dinv64 = jnp.broadcast_to(dinv, (BLK, HID))
    dinv_ref[...] = dinv64
    hs_ref[...] = jnp.broadcast_to(_pad_fw(dinv64 * h_ref[...]), (NC, BLK, FW))


_scale = pl.pallas_call(
    _scale_body,
    grid=(N_PAD // BLK,),
    in_specs=[
        pl.BlockSpec((2, BLK, DW), lambda i: (0, i, 0)),
        pl.BlockSpec((BLK, HID), lambda i: (i, 0)),
    ],
    out_specs=[
        pl.BlockSpec((BLK, HID), lambda i: (i, 0)),
        pl.BlockSpec((NC, BLK, FW), lambda i: (0, i, 0)),
    ],
    out_shape=[
        jax.ShapeDtypeStruct((N_PAD, HID), _f32),
        jax.ShapeDtypeStruct((NC, N_PAD, FW), _f32),
    ],
)


def _eye64():
    r = lax.broadcasted_iota(jnp.int32, (HID, HID), 0)
    q = lax.broadcasted_iota(jnp.int32, (HID, HID), 1)
    return (r == q).astype(_f32)


def _res_block(parts_ref, hs_ref, x0_ref, dinv_ref, w_ref, beta):
    dinv = dinv_ref[...]
    agg = parts_ref[0, :, :HID] + parts_ref[1, :, :HID] + hs_ref[0, :, :HID]
    t = dinv * agg
    hr = (1.0 - ALPHA) * t + ALPHA * x0_ref[...]
    wp = beta * w_ref[...] + (1.0 - beta) * _eye64()
    return jnp.maximum(jnp.dot(hr, wp, preferred_element_type=_f32), 0.0)


def _layer_body(parts_ref, hs_ref, x0_ref, dinv_ref, w_ref, o_ref, *, beta):
    h = _res_block(parts_ref, hs_ref, x0_ref, dinv_ref, w_ref, beta)
    o_ref[...] = jnp.broadcast_to(_pad_fw(dinv_ref[...] * h), (NC, BLK, FW))


def _last_body(parts_ref, hs_ref, x0_ref, dinv_ref, w_ref, w1_ref, b1_ref,
               o_ref, *, beta):
    h = _res_block(parts_ref, hs_ref, x0_ref, dinv_ref, w_ref, beta)
    o = jnp.dot(h, w1_ref[...], preferred_element_type=_f32) + b1_ref[...]
    m = jnp.max(o, axis=-1, keepdims=True)
    sh = o - m
    lse = jnp.log(jnp.sum(jnp.exp(sh), axis=-1, keepdims=True))
    o_ref[...] = sh - lse


def _make_layer(beta):
    return pl.pallas_call(
        functools.partial(_layer_body, beta=beta),
        grid=(N_PAD // BLK,),
        in_specs=[
            pl.BlockSpec((2, BLK, FW), lambda i: (0, i, 0)),
            pl.BlockSpec((NC, BLK, FW), lambda i: (0, i, 0)),
            pl.BlockSpec((BLK, HID), lambda i: (i, 0)),
            pl.BlockSpec((BLK, HID), lambda i: (i, 0)),
            pl.BlockSpec((HID, HID), lambda i: (0, 0)),
        ],
        out_specs=pl.BlockSpec((NC, BLK, FW), lambda i: (0, i, 0)),
        out_shape=jax.ShapeDtypeStruct((NC, N_PAD, FW), _f32),
    )


def _make_last(beta):
    return pl.pallas_call(
        functools.partial(_last_body, beta=beta),
        grid=(N_PAD // BLK,),
        in_specs=[
            pl.BlockSpec((2, BLK, FW), lambda i: (0, i, 0)),
            pl.BlockSpec((NC, BLK, FW), lambda i: (0, i, 0)),
            pl.BlockSpec((BLK, HID), lambda i: (i, 0)),
            pl.BlockSpec((BLK, HID), lambda i: (i, 0)),
            pl.BlockSpec((HID, HID), lambda i: (0, 0)),
            pl.BlockSpec((HID, C), lambda i: (0, 0)),
            pl.BlockSpec((1, C), lambda i: (0, 0)),
        ],
        out_specs=pl.BlockSpec((BLK, C), lambda i: (i, 0)),
        out_shape=jax.ShapeDtypeStruct((N_PAD, C), _f32),
    )


_BETAS = [float(np.log(THETA / (l + 1) + 1.0)) for l in range(NUM_LAYERS)]
_layers = [_make_layer(b) for b in _BETAS[:-1]]
_last = _make_last(_BETAS[-1])


@jax.jit
def _forward(x, edge_index, W0, b0, W1, b1, convW):
    xp = jnp.zeros((N_PAD, F_IN), _f32).at[:N].set(x)
    src = edge_index[0].astype(jnp.int32)
    dst = edge_index[1].astype(jnp.int32)
    pad = E_PAD - E
    fill = jnp.full((pad,), N, jnp.int32)
    src_p = jnp.concatenate([src, fill]).reshape(NC, NS, CHUNKS_W, CHUNK)
    dst_p = jnp.concatenate([dst, fill]).reshape(NC, NS, CHUNKS_W, CHUNK)

    sc_deg, sc_spmm = _sc_kernels()
    degp = sc_deg(dst_p)
    h0 = _lin0(xp, W0, b0.reshape(1, HID))
    dinv64, hs = _scale(degp, h0)
    for l in range(NUM_LAYERS - 1):
        parts = sc_spmm(hs, src_p, dst_p)
        hs = _layers[l](parts, hs, h0, dinv64, convW[l])
    parts = sc_spmm(hs, src_p, dst_p)
    out = _last(parts, hs, h0, dinv64, convW[NUM_LAYERS - 1], W1,
                b1.reshape(1, C))
    return out[:N]


def kernel(x, edge_index, W0, b0, W1, b1, convW):
    return _forward(x, edge_index, W0, b0, W1, b1, convW)
